# baseline (speedup 1.0000x reference)
import jax
import jax.numpy as jnp
from jax import lax
from jax.experimental import pallas as pl
from jax.experimental.pallas import tpu as pltpu
from jax.experimental.pallas import tpu_sc as plsc

D = 1024
B = 4
T = 8192
NC = 2
NS = 16
NW = NC * NS
NB = (B * T) // NW
C = 32
NCHUNK = NB // C


def _body(x_hbm, tok_hbm, pos_hbm, out_hbm, idx_v, tok_buf, pos_buf, sem):
    wid = lax.axis_index("s") * NC + lax.axis_index("c")
    base = wid * NB
    t0 = lax.rem(base, T)
    pltpu.sync_copy(x_hbm.at[wid], idx_v)

    def chunk_body(c, carry):
        row0 = c * C
        pltpu.async_copy(tok_hbm.at[idx_v.at[c]], tok_buf, sem).wait()
        pltpu.sync_copy(pos_hbm.at[pl.ds(t0 + row0, C)], pos_buf)

        def add_row(j, carry2):
            for i in range(D // 16):
                sl = pl.ds(i * 16, 16)
                tok_buf[j, sl] = tok_buf[j, sl] + pos_buf[j, sl]
            return carry2

        lax.fori_loop(0, C, add_row, 0)
        pltpu.sync_copy(tok_buf, out_hbm.at[pl.ds(base + row0, C)])
        return carry

    lax.fori_loop(0, NCHUNK, chunk_body, 0)


def kernel(x, token_table, pos_table):
    xf = x.reshape(NW, NCHUNK, C).astype(jnp.int32)
    mesh = plsc.VectorSubcoreMesh(core_axis_name="c", subcore_axis_name="s")
    k = pl.kernel(
        _body,
        out_type=jax.ShapeDtypeStruct((B * T, D), jnp.float32),
        mesh=mesh,
        scratch_types=[
            pltpu.VMEM((NCHUNK, C), jnp.int32),
            pltpu.VMEM((C, D), jnp.float32),
            pltpu.VMEM((C, D), jnp.float32),
            pltpu.SemaphoreType.DMA,
        ],
    )
    out = k(xf, token_table, pos_table)
    return out.reshape(B, T, D)


# speedup vs baseline: 2.0239x; 2.0239x over previous
import jax
import jax.numpy as jnp
from jax import lax
from jax.experimental import pallas as pl
from jax.experimental.pallas import tpu as pltpu
from jax.experimental.pallas import tpu_sc as plsc

D = 1024
B = 4
T = 8192
NC = 2
NS = 16
NW = NC * NS
TPW = T // NW
P = 4
NCH = TPW // P
NBUF = 4
RPB = B * P


def _body(x_hbm, tok_hbm, pos_hbm, out_hbm, idx_v, tok_buf, pos_buf, *sems):
    sem_g = sems[0:4]
    sem_p = sems[4:8]
    sem_w = sems[8:12]
    wid = lax.axis_index("s") * NC + lax.axis_index("c")
    t0 = wid * TPW
    pltpu.sync_copy(x_hbm.at[wid], idx_v)

    def issue_round(r, p):
        pltpu.async_copy(tok_hbm.at[idx_v.at[r]], tok_buf.at[p], sem_g[p])
        pltpu.async_copy(pos_hbm.at[pl.ds(t0 + r * P, P)], pos_buf.at[p], sem_p[p])

    def wait_round(r, p):
        pltpu.make_async_copy(tok_hbm.at[idx_v.at[r]], tok_buf.at[p], sem_g[p]).wait()
        pltpu.make_async_copy(pos_hbm.at[pl.ds(t0 + r * P, P)], pos_buf.at[p], sem_p[p]).wait()

    def issue_writes(r, p):
        for b in range(B):
            pltpu.async_copy(tok_buf.at[p, pl.ds(b * P, P)],
                             out_hbm.at[pl.ds(b * T + t0 + r * P, P)], sem_w[p])

    def drain_writes(r, p):
        for b in range(B):
            pltpu.make_async_copy(tok_buf.at[p, pl.ds(b * P, P)],
                                  out_hbm.at[pl.ds(b * T + t0 + r * P, P)], sem_w[p]).wait()

    issue_round(0, 0)

    def outer(o, carry):
        for p in range(NBUF):
            r = NBUF * o + p
            pn = (p + 1) % NBUF

            @pl.when(r >= NBUF - 1)
            def _():
                drain_writes(r - (NBUF - 1), pn)

            @pl.when(r + 1 < NCH)
            def _():
                issue_round(r + 1, pn)

            wait_round(r, p)

            def add_i(i, c2):
                sl = pl.ds(i * 16, 16)
                for j in range(P):
                    pv = pos_buf[p, j, sl]
                    for b in range(B):
                        row = b * P + j
                        tok_buf[p, row, sl] = tok_buf[p, row, sl] + pv
                return c2

            lax.fori_loop(0, D // 16, add_i, 0)
            issue_writes(r, p)
        return carry

    lax.fori_loop(0, NCH // NBUF, outer, 0)
    for r in range(NCH - (NBUF - 1), NCH):
        drain_writes(r, r % NBUF)


def kernel(x, token_table, pos_table):
    xf = (x.astype(jnp.int32)
          .reshape(B, NW, NCH, P)
          .transpose(1, 2, 0, 3)
          .reshape(NW, NCH, RPB))
    mesh = plsc.VectorSubcoreMesh(core_axis_name="c", subcore_axis_name="s")
    k = pl.kernel(
        _body,
        out_type=jax.ShapeDtypeStruct((B * T, D), jnp.float32),
        mesh=mesh,
        scratch_types=[
            pltpu.VMEM((NCH, RPB), jnp.int32),
            pltpu.VMEM((NBUF, RPB, D), jnp.float32),
            pltpu.VMEM((NBUF, P, D), jnp.float32),
        ] + [pltpu.SemaphoreType.DMA] * 12,
    )
    out = k(xf, token_table, pos_table)
    return out.reshape(B, T, D)
